# baseline (speedup 1.0000x reference)
import functools

import jax
import jax.numpy as jnp
from jax import lax
from jax.experimental import pallas as pl
from jax.experimental.pallas import tpu as pltpu
from jax.experimental.pallas import tpu_sc as plsc

RATIO = 16
NC, NS, L = 2, 16, 16
NW = NC * NS


@functools.lru_cache(maxsize=None)
def _make_sc_call(n_groups: int, d: int):
    assert n_groups % (NW * L) == 0 and d % L == 0
    g_per_w = n_groups // NW
    mesh = plsc.VectorSubcoreMesh(
        core_axis_name="c", subcore_axis_name="s",
        num_cores=NC, num_subcores=NS)

    @functools.partial(
        pl.kernel,
        out_type=[
            jax.ShapeDtypeStruct((n_groups, d), jnp.float32),
            jax.ShapeDtypeStruct((n_groups,), jnp.int32),
        ],
        mesh=mesh,
        scratch_types=[
            pltpu.VMEM((g_per_w * RATIO,), jnp.int32),
            pltpu.VMEM((g_per_w,), jnp.int32),
            pltpu.VMEM((g_per_w,), jnp.int32),
            pltpu.VMEM((g_per_w, d), jnp.float32),
            pltpu.SemaphoreType.DMA,
        ],
    )
    def sc_call(mask_hbm, table_hbm, out_hbm, flags_hbm,
                mask_v, gid_v, flags_v, rows_v, sem):
        wid = lax.axis_index("s") * NC + lax.axis_index("c")
        gbase = wid * g_per_w
        pltpu.sync_copy(mask_hbm.at[pl.ds(gbase * RATIO, g_per_w * RATIO)],
                        mask_v)
        for vb in range(g_per_w // L):
            acc = mask_v[pl.ds(vb * L, L)]
            for j in range(1, RATIO):
                acc = acc & mask_v[pl.ds(j * g_per_w + vb * L, L)]
            flags_v[pl.ds(vb * L, L)] = acc
            gid_v[pl.ds(vb * L, L)] = 1 - acc
        pltpu.async_copy(table_hbm.at[gid_v], rows_v, sem).wait()
        pltpu.sync_copy(rows_v, out_hbm.at[pl.ds(gbase, g_per_w)])
        pltpu.sync_copy(flags_v, flags_hbm.at[pl.ds(gbase, g_per_w)])

    return sc_call


def kernel(token_ids, padding_mask, embeds):
    B, Sl = padding_mask.shape
    d = embeds.shape[1]
    Sg = Sl // RATIO
    n = B * Sg
    mask_i32 = (padding_mask.reshape(NW, n // NW, RATIO)
                .transpose(0, 2, 1).astype(jnp.int32).reshape(n * RATIO))
    out_flat, flags = _make_sc_call(n, d)(mask_i32, embeds)
    return out_flat.reshape(B, Sg, d), flags.reshape(B, Sg).astype(jnp.bool_)


# speedup vs baseline: 3.5314x; 3.5314x over previous
import functools

import jax
import jax.numpy as jnp
from jax import lax
from jax.experimental import pallas as pl
from jax.experimental.pallas import tpu as pltpu
from jax.experimental.pallas import tpu_sc as plsc

RATIO = 16
NC, NS, L = 2, 16, 16
NW = NC * NS


@functools.lru_cache(maxsize=None)
def _make_sc_call(n_groups: int, d: int):
    assert n_groups % (NW * L) == 0 and d % L == 0
    g_per_w = n_groups // NW
    mesh = plsc.VectorSubcoreMesh(
        core_axis_name="c", subcore_axis_name="s",
        num_cores=NC, num_subcores=NS)

    @functools.partial(
        pl.kernel,
        out_type=[
            jax.ShapeDtypeStruct((n_groups, d), jnp.float32),
            jax.ShapeDtypeStruct((n_groups,), jnp.int32),
        ],
        mesh=mesh,
        scratch_types=[
            pltpu.VMEM((g_per_w * RATIO,), jnp.int32),
            pltpu.VMEM((g_per_w,), jnp.int32),
            pltpu.VMEM((2, d), jnp.float32),
            pltpu.SemaphoreType.DMA,
            pltpu.SemaphoreType.DMA,
        ],
    )
    def sc_call(mask_hbm, table_hbm, out_hbm, flags_hbm,
                mask_v, flags_v, table_v, sem, out_sem):
        wid = lax.axis_index("s") * NC + lax.axis_index("c")
        gbase = wid * g_per_w
        cp_mask = pltpu.async_copy(
            mask_hbm.at[pl.ds(gbase * RATIO, g_per_w * RATIO)], mask_v, sem)
        cp_tab = pltpu.async_copy(table_hbm, table_v, sem)
        cp_mask.wait()
        for vb in range(g_per_w // L):
            acc = mask_v[pl.ds(vb * L, L)]
            for j in range(1, RATIO):
                acc = acc & mask_v[pl.ds(j * g_per_w + vb * L, L)]
            flags_v[pl.ds(vb * L, L)] = acc
        cp_tab.wait()
        rows = []
        for vb in range(g_per_w // L):
            fl = flags_v[pl.ds(vb * L, L)]
            for g in range(L):
                r = vb * L + g
                rid = jnp.where(fl[g] != 0, 0, 1)
                rows.append(pltpu.async_copy(table_v.at[rid],
                                             out_hbm.at[gbase + r], out_sem))
        cp_flags = pltpu.async_copy(flags_v,
                                    flags_hbm.at[pl.ds(gbase, g_per_w)], sem)
        for cp in rows:
            cp.wait()
        cp_flags.wait()

    return sc_call


def kernel(token_ids, padding_mask, embeds):
    B, Sl = padding_mask.shape
    d = embeds.shape[1]
    Sg = Sl // RATIO
    n = B * Sg
    mask_i32 = (padding_mask.reshape(NW, n // NW, RATIO)
                .transpose(0, 2, 1).astype(jnp.int32).reshape(n * RATIO))
    out_flat, flags = _make_sc_call(n, d)(mask_i32, embeds)
    return out_flat.reshape(B, Sg, d), flags.reshape(B, Sg).astype(jnp.bool_)


# speedup vs baseline: 3.7906x; 1.0734x over previous
import functools

import jax
import jax.numpy as jnp
from jax import lax
from jax.experimental import pallas as pl
from jax.experimental.pallas import tpu as pltpu
from jax.experimental.pallas import tpu_sc as plsc

RATIO = 16
NC, NS, L = 2, 16, 16
NW = NC * NS


@functools.lru_cache(maxsize=None)
def _make_sc_call(n_groups: int, d: int):
    assert n_groups % (NW * L) == 0 and d % L == 0
    g_per_w = n_groups // NW
    mesh = plsc.VectorSubcoreMesh(
        core_axis_name="c", subcore_axis_name="s",
        num_cores=NC, num_subcores=NS)

    @functools.partial(
        pl.kernel,
        out_type=[
            jax.ShapeDtypeStruct((n_groups, d), jnp.float32),
            jax.ShapeDtypeStruct((n_groups,), jnp.int32),
        ],
        mesh=mesh,
        scratch_types=[
            pltpu.VMEM((g_per_w * RATIO,), jnp.int32),
            pltpu.VMEM((g_per_w,), jnp.int32),
            pltpu.VMEM((2, d), jnp.float32),
            pltpu.SemaphoreType.DMA,
            pltpu.SemaphoreType.DMA,
        ],
    )
    def sc_call(mask_hbm, table_hbm, out_hbm, flags_hbm,
                mask_v, flags_v, table_v, sem, out_sem):
        wid = lax.axis_index("s") * NC + lax.axis_index("c")
        gbase = wid * g_per_w
        cp_mask = pltpu.async_copy(
            mask_hbm.at[pl.ds(gbase * RATIO, g_per_w * RATIO)], mask_v, sem)
        cp_tab = pltpu.async_copy(table_hbm, table_v, sem)
        cp_mask.wait()
        cp_tab.wait()

        @pl.loop(0, g_per_w // L)
        def _row_block(vb):
            base = vb * L
            acc = mask_v[pl.ds(base, L)]
            for j in range(1, RATIO):
                acc = acc & mask_v[pl.ds(j * g_per_w + base, L)]
            flags_v[pl.ds(base, L)] = acc
            for g in range(L):
                rid = jnp.where(acc[g] != 0, 0, 1)
                pltpu.async_copy(table_v.at[rid],
                                 out_hbm.at[gbase + base + g], out_sem)

        cp_flags = pltpu.async_copy(flags_v,
                                    flags_hbm.at[pl.ds(gbase, g_per_w)], sem)

        @pl.loop(0, g_per_w)
        def _drain(r):
            pltpu.make_async_copy(table_v.at[1],
                                  out_hbm.at[gbase + r], out_sem).wait()

        cp_flags.wait()

    return sc_call


def kernel(token_ids, padding_mask, embeds):
    B, Sl = padding_mask.shape
    d = embeds.shape[1]
    Sg = Sl // RATIO
    n = B * Sg
    mask_i32 = (padding_mask.reshape(NW, n // NW, RATIO)
                .transpose(0, 2, 1).astype(jnp.int32).reshape(n * RATIO))
    out_flat, flags = _make_sc_call(n, d)(mask_i32, embeds)
    return out_flat.reshape(B, Sg, d), flags.reshape(B, Sg).astype(jnp.bool_)


# speedup vs baseline: 3.9109x; 1.0317x over previous
import functools

import jax
import jax.numpy as jnp
from jax import lax
from jax.experimental import pallas as pl
from jax.experimental.pallas import tpu as pltpu
from jax.experimental.pallas import tpu_sc as plsc

RATIO = 16
NC, NS, L = 2, 16, 16
NW = NC * NS


@functools.lru_cache(maxsize=None)
def _make_sc_call(n_groups: int, d: int):
    assert n_groups % (NW * L) == 0 and d % L == 0
    g_per_w = n_groups // NW
    mesh = plsc.VectorSubcoreMesh(
        core_axis_name="c", subcore_axis_name="s",
        num_cores=NC, num_subcores=NS)

    @functools.partial(
        pl.kernel,
        out_type=[
            jax.ShapeDtypeStruct((n_groups, d), jnp.float32),
            jax.ShapeDtypeStruct((n_groups,), jnp.int32),
        ],
        mesh=mesh,
        scratch_types=[
            pltpu.VMEM((g_per_w * RATIO,), jnp.int32),
            pltpu.VMEM((g_per_w,), jnp.int32),
            pltpu.VMEM((2, d), jnp.float32),
            pltpu.SMEM((g_per_w,), jnp.int32),
            pltpu.SemaphoreType.DMA,
            pltpu.SemaphoreType.DMA,
        ],
    )
    def sc_call(mask_hbm, table_hbm, out_hbm, flags_hbm,
                mask_v, flags_v, table_v, flags_s, sem, out_sem):
        wid = lax.axis_index("s") * NC + lax.axis_index("c")
        gbase = wid * g_per_w
        cp_mask = pltpu.async_copy(
            mask_hbm.at[pl.ds(gbase * RATIO, g_per_w * RATIO)], mask_v, sem)
        cp_tab = pltpu.async_copy(table_hbm, table_v, sem)
        cp_mask.wait()
        cp_tab.wait()

        @pl.loop(0, g_per_w // L)
        def _flag_block(vb):
            base = vb * L
            acc = mask_v[pl.ds(base, L)]
            for j in range(1, RATIO):
                acc = acc & mask_v[pl.ds(j * g_per_w + base, L)]
            flags_v[pl.ds(base, L)] = acc
            for g in range(L):
                flags_s[base + g] = acc[g]

        cp_flags = pltpu.async_copy(flags_v,
                                    flags_hbm.at[pl.ds(gbase, g_per_w)], sem)

        @pl.loop(0, g_per_w)
        def _row(r):
            rid = jnp.where(flags_s[r] != 0, 0, 1)
            pltpu.async_copy(table_v.at[rid],
                             out_hbm.at[gbase + r], out_sem)

        @pl.loop(0, g_per_w)
        def _drain(r):
            pltpu.make_async_copy(table_v.at[1],
                                  out_hbm.at[gbase + r], out_sem).wait()

        cp_flags.wait()

    return sc_call


def kernel(token_ids, padding_mask, embeds):
    B, Sl = padding_mask.shape
    d = embeds.shape[1]
    Sg = Sl // RATIO
    n = B * Sg
    mask_i32 = (padding_mask.reshape(NW, n // NW, RATIO)
                .transpose(0, 2, 1).astype(jnp.int32).reshape(n * RATIO))
    out_flat, flags = _make_sc_call(n, d)(mask_i32, embeds)
    return out_flat.reshape(B, Sg, d), flags.reshape(B, Sg).astype(jnp.bool_)


# speedup vs baseline: 9.0950x; 2.3256x over previous
import functools

import jax
import jax.numpy as jnp
from jax.experimental import pallas as pl

RATIO = 16


def _body(mask_ref, emb_ref, out_ref, flag_ref):
    flags = jnp.min(mask_ref[...], axis=1)
    flag_ref[0, 0, :] = flags
    keep = (1 - flags).astype(jnp.float32)
    out_ref[...] = keep[:, None] * emb_ref[1, :][None, :]


@functools.lru_cache(maxsize=None)
def _make_tc_call(n: int, d: int, blk: int):
    grid = n // blk
    return pl.pallas_call(
        _body,
        grid=(grid,),
        in_specs=[
            pl.BlockSpec((blk, RATIO), lambda i: (i, 0)),
            pl.BlockSpec((2, d), lambda i: (0, 0)),
        ],
        out_specs=[
            pl.BlockSpec((blk, d), lambda i: (i, 0)),
            pl.BlockSpec((1, 1, blk), lambda i: (i, 0, 0)),
        ],
        out_shape=[
            jax.ShapeDtypeStruct((n, d), jnp.float32),
            jax.ShapeDtypeStruct((grid, 1, blk), jnp.int32),
        ],
    )


def kernel(token_ids, padding_mask, embeds):
    B, Sl = padding_mask.shape
    d = embeds.shape[1]
    Sg = Sl // RATIO
    n = B * Sg
    mask_i32 = padding_mask.astype(jnp.int32).reshape(n, RATIO)
    out_flat, flags = _make_tc_call(n, d, 256)(mask_i32, embeds)
    return (out_flat.reshape(B, Sg, d),
            flags.reshape(B, Sg).astype(jnp.bool_))


# speedup vs baseline: 9.3622x; 1.0294x over previous
import functools

import jax
import jax.numpy as jnp
from jax.experimental import pallas as pl

RATIO = 16


def _body(mask_ref, emb_ref, out_ref, flag_ref):
    flags = jnp.all(mask_ref[...], axis=1)
    flag_ref[0, 0, :] = flags
    keep = 1.0 - flags.astype(jnp.float32)
    out_ref[...] = keep[:, None] * emb_ref[1, :][None, :]


@functools.lru_cache(maxsize=None)
def _make_tc_call(n: int, d: int, blk: int):
    grid = n // blk
    return pl.pallas_call(
        _body,
        grid=(grid,),
        in_specs=[
            pl.BlockSpec((blk, RATIO), lambda i: (i, 0)),
            pl.BlockSpec((2, d), lambda i: (0, 0)),
        ],
        out_specs=[
            pl.BlockSpec((blk, d), lambda i: (i, 0)),
            pl.BlockSpec((1, 1, blk), lambda i: (i, 0, 0)),
        ],
        out_shape=[
            jax.ShapeDtypeStruct((n, d), jnp.float32),
            jax.ShapeDtypeStruct((grid, 1, blk), jnp.bool_),
        ],
    )


def kernel(token_ids, padding_mask, embeds):
    B, Sl = padding_mask.shape
    d = embeds.shape[1]
    Sg = Sl // RATIO
    n = B * Sg
    out_flat, flags = _make_tc_call(n, d, 256)(
        padding_mask.reshape(n, RATIO), embeds)
    return out_flat.reshape(B, Sg, d), flags.reshape(B, Sg)


# speedup vs baseline: 11.2231x; 1.1988x over previous
import functools

import jax
import jax.numpy as jnp
from jax.experimental import pallas as pl

RATIO = 16


def _body(mask_ref, emb_ref, out_ref, flag_ref):
    flags = jnp.all(mask_ref[...], axis=1)
    flag_ref[0, 0, :] = flags
    keep = 1.0 - flags.astype(jnp.float32)
    out_ref[...] = keep[:, None] * emb_ref[1, :][None, :]


@functools.lru_cache(maxsize=None)
def _make_tc_call(n: int, d: int, blk: int):
    grid = n // blk
    return pl.pallas_call(
        _body,
        grid=(grid,),
        in_specs=[
            pl.BlockSpec((blk, RATIO), lambda i: (i, 0)),
            pl.BlockSpec((2, d), lambda i: (0, 0)),
        ],
        out_specs=[
            pl.BlockSpec((blk, d), lambda i: (i, 0)),
            pl.BlockSpec((1, 1, blk), lambda i: (i, 0, 0)),
        ],
        out_shape=[
            jax.ShapeDtypeStruct((n, d), jnp.float32),
            jax.ShapeDtypeStruct((grid, 1, blk), jnp.bool_),
        ],
    )


def kernel(token_ids, padding_mask, embeds):
    B, Sl = padding_mask.shape
    d = embeds.shape[1]
    Sg = Sl // RATIO
    n = B * Sg
    out_flat, flags = _make_tc_call(n, d, 512)(
        padding_mask.reshape(n, RATIO), embeds)
    return out_flat.reshape(B, Sg, d), flags.reshape(B, Sg)


# speedup vs baseline: 12.1272x; 1.0806x over previous
import functools

import jax
import jax.numpy as jnp
from jax.experimental import pallas as pl

RATIO = 16


def _body(mask_ref, emb_ref, out_ref, flag_ref):
    flags = jnp.all(mask_ref[...], axis=1)
    flag_ref[0, 0, :] = flags
    keep = 1.0 - flags.astype(jnp.float32)
    out_ref[...] = keep[:, None] * emb_ref[1, :][None, :]


@functools.lru_cache(maxsize=None)
def _make_tc_call(n: int, d: int, blk: int):
    grid = n // blk
    return pl.pallas_call(
        _body,
        grid=(grid,),
        in_specs=[
            pl.BlockSpec((blk, RATIO), lambda i: (i, 0)),
            pl.BlockSpec((2, d), lambda i: (0, 0)),
        ],
        out_specs=[
            pl.BlockSpec((blk, d), lambda i: (i, 0)),
            pl.BlockSpec((1, 1, blk), lambda i: (i, 0, 0)),
        ],
        out_shape=[
            jax.ShapeDtypeStruct((n, d), jnp.float32),
            jax.ShapeDtypeStruct((grid, 1, blk), jnp.bool_),
        ],
    )


def kernel(token_ids, padding_mask, embeds):
    B, Sl = padding_mask.shape
    d = embeds.shape[1]
    Sg = Sl // RATIO
    n = B * Sg
    out_flat, flags = _make_tc_call(n, d, 1024)(
        padding_mask.reshape(n, RATIO), embeds)
    return out_flat.reshape(B, Sg, d), flags.reshape(B, Sg)
